# Initial kernel scaffold; baseline (speedup 1.0000x reference)
import jax
import jax.numpy as jnp
from jax import lax
from jax.experimental import pallas as pl
from jax.experimental.pallas import tpu as pltpu
from jax.experimental.pallas import tpu_sc as plsc

N = 10000
NUM_GRAPHS = 64
OUT = 10
NC = 2
NS = 16
NW = NC * NS
CH = 128
NBUF = 4
NPAD = 10240
RPT = NPAD // NS
CNTW = 16


def _make_edge_agg(d, nchunk, with_cnt):
    ept = nchunk * CH
    mesh = plsc.VectorSubcoreMesh(core_axis_name="c", subcore_axis_name="s")

    out_type = [jax.ShapeDtypeStruct((NC, NPAD, d), jnp.float32)]
    scratch = [
        pltpu.VMEM((ept,), jnp.int32),
        pltpu.VMEM((nchunk, CH), jnp.int32),
        pltpu.VMEM((NBUF, CH, d), jnp.float32),
        pltpu.VMEM((CH, d), jnp.float32),
        pltpu.VMEM_SHARED((NPAD, d), jnp.float32),
    ] + [pltpu.SemaphoreType.DMA] * NBUF
    if with_cnt:
        out_type.append(jax.ShapeDtypeStruct((NC, NPAD, CNTW), jnp.float32))
        scratch += [
            pltpu.VMEM((CH, CNTW), jnp.float32),
            pltpu.VMEM((CH, CNTW), jnp.float32),
            pltpu.VMEM_SHARED((NPAD, CNTW), jnp.float32),
        ]

    def body(table, srcs, dsts, zeros_hbm, *rest):
        if with_cnt:
            ones_hbm, zeros16_hbm, out_p, out_c = rest[:4]
            rest = rest[4:]
        else:
            out_p = rest[0]
            rest = rest[1:]
        src_v, dst_v, rows_v, zero_v, agg_sh = rest[:5]
        sems = rest[5:5 + NBUF]
        if with_cnt:
            ones_v, zcnt_v, cnt_sh = rest[5 + NBUF:]

        cid = lax.axis_index("c")
        sid = lax.axis_index("s")
        wid = cid * NS + sid
        row0 = sid * RPT

        pltpu.sync_copy(srcs.at[wid], src_v)
        pltpu.sync_copy(dsts.at[wid], dst_v)
        pltpu.sync_copy(zeros_hbm, zero_v)
        for j in range(RPT // CH):
            pltpu.sync_copy(zero_v, agg_sh.at[pl.ds(row0 + j * CH, CH)])
        if with_cnt:
            pltpu.sync_copy(ones_hbm, ones_v)
            pltpu.sync_copy(zeros16_hbm, zcnt_v)
            for j in range(RPT // CH):
                pltpu.sync_copy(zcnt_v, cnt_sh.at[pl.ds(row0 + j * CH, CH)])
        plsc.subcore_barrier()

        def start_gather(chunk, b):
            pltpu.async_copy(
                table.at[src_v.at[pl.ds(chunk * CH, CH)]], rows_v.at[b],
                sems[b])

        def wait_gather(b):
            pltpu.make_async_copy(
                table.at[src_v.at[pl.ds(0, CH)]], rows_v.at[b],
                sems[b]).wait()

        def scatter(chunk, b):
            pltpu.sync_copy(rows_v.at[b], agg_sh.at[dst_v.at[chunk]],
                            add=True)
            if with_cnt:
                pltpu.sync_copy(ones_v, cnt_sh.at[dst_v.at[chunk]], add=True)

        for b in range(NBUF):
            start_gather(b, b)

        @pl.loop(0, nchunk - NBUF, step=NBUF)
        def _(j0):
            for b in range(NBUF):
                wait_gather(b)
                scatter(j0 + b, b)
                start_gather(j0 + NBUF + b, b)

        for b in range(NBUF):
            wait_gather(b)
            scatter(nchunk - NBUF + b, b)

        plsc.subcore_barrier()
        pltpu.sync_copy(agg_sh.at[pl.ds(row0, RPT)],
                        out_p.at[cid, pl.ds(row0, RPT)])
        if with_cnt:
            pltpu.sync_copy(cnt_sh.at[pl.ds(row0, RPT)],
                            out_c.at[cid, pl.ds(row0, RPT)])

    return pl.kernel(body, out_type=tuple(out_type), mesh=mesh,
                     scratch_types=scratch)


BN = 2048


def _stage_in_body(x_ref, wl_ref, wr_ref, t_ref, r_ref):
    xb = x_ref[...]
    t_ref[...] = jnp.dot(xb, wl_ref[...], preferred_element_type=jnp.float32)
    r_ref[...] = jnp.dot(xb, wr_ref[...], preferred_element_type=jnp.float32)


def _stage_in(x_pad, wlt, wrt):
    din, dout = wlt.shape
    return pl.pallas_call(
        _stage_in_body,
        grid=(NPAD // BN,),
        in_specs=[
            pl.BlockSpec((BN, din), lambda i: (i, 0)),
            pl.BlockSpec((din, dout), lambda i: (0, 0)),
            pl.BlockSpec((din, dout), lambda i: (0, 0)),
        ],
        out_specs=[
            pl.BlockSpec((BN, dout), lambda i: (i, 0)),
            pl.BlockSpec((BN, dout), lambda i: (i, 0)),
        ],
        out_shape=[
            jax.ShapeDtypeStruct((NPAD, dout), jnp.float32),
            jax.ShapeDtypeStruct((NPAD, dout), jnp.float32),
        ],
    )(x_pad, wlt, wrt)


def _stage_mid_body(p0, p1, c0, c1, r, b, wl, wr, t_ref, r_ref):
    cnt = jnp.maximum(c0[...] + c1[...], 1.0)
    h = (p0[...] + p1[...]) / cnt + b[...] + r[...]
    h = jnp.maximum(h, 0.0)
    t_ref[...] = jnp.dot(h, wl[...], preferred_element_type=jnp.float32)
    r_ref[...] = jnp.dot(h, wr[...], preferred_element_type=jnp.float32)


def _stage_mid(p0, p1, c0, c1, r, bias, wlt, wrt):
    din, dout = wlt.shape
    col = pl.BlockSpec((BN, 1), lambda i: (i, 0))
    blk = pl.BlockSpec((BN, din), lambda i: (i, 0))
    return pl.pallas_call(
        _stage_mid_body,
        grid=(NPAD // BN,),
        in_specs=[blk, blk, col, col, blk,
                  pl.BlockSpec((1, din), lambda i: (0, 0)),
                  pl.BlockSpec((din, dout), lambda i: (0, 0)),
                  pl.BlockSpec((din, dout), lambda i: (0, 0))],
        out_specs=[
            pl.BlockSpec((BN, dout), lambda i: (i, 0)),
            pl.BlockSpec((BN, dout), lambda i: (i, 0)),
        ],
        out_shape=[
            jax.ShapeDtypeStruct((NPAD, dout), jnp.float32),
            jax.ShapeDtypeStruct((NPAD, dout), jnp.float32),
        ],
    )(p0, p1, c0, c1, r, bias, wlt, wrt)


def _stage_out_body(p0, p1, c0, c1, r, bcol, b3, w1, b1, w2, b2, w3, b3b,
                    out_ref, gsum, gcnt):
    i = pl.program_id(0)

    @pl.when(i == 0)
    def _():
        gsum[...] = jnp.zeros_like(gsum)
        gcnt[...] = jnp.zeros_like(gcnt)

    cnt = jnp.maximum(c0[...] + c1[...], 1.0)
    h = (p0[...] + p1[...]) / cnt + b3[...] + r[...]
    gids = lax.broadcasted_iota(jnp.int32, (1, NUM_GRAPHS), 1)
    maskt = (bcol[...] == gids).astype(jnp.float32)
    dn = (((0,), (0,)), ((), ()))
    gsum[...] += lax.dot_general(maskt, h, dn,
                                 preferred_element_type=jnp.float32)
    ones = jnp.ones((BN, 1), jnp.float32)
    gcnt[...] += lax.dot_general(maskt, ones, dn,
                                 preferred_element_type=jnp.float32)

    @pl.when(i == pl.num_programs(0) - 1)
    def _():
        g = gsum[...] / jnp.maximum(gcnt[...], 1.0)
        z = jnp.dot(g, w1[...], preferred_element_type=jnp.float32) + b1[...]
        z = jnp.maximum(z, 0.0)
        z = jnp.dot(z, w2[...], preferred_element_type=jnp.float32) + b2[...]
        z = jnp.maximum(z, 0.0)
        out_ref[...] = (jnp.dot(z, w3[...],
                                preferred_element_type=jnp.float32) + b3b[...])


def _stage_out(p0, p1, c0, c1, r, bcol, b3, w1t, b1, w2t, b2, w3t, b3b):
    din = p0.shape[1]
    col = pl.BlockSpec((BN, 1), lambda i: (i, 0))
    blk = pl.BlockSpec((BN, din), lambda i: (i, 0))

    def full(a):
        return pl.BlockSpec(a.shape, lambda i: tuple(0 for _ in a.shape))

    return pl.pallas_call(
        _stage_out_body,
        grid=(NPAD // BN,),
        in_specs=[blk, blk, col, col, blk, col,
                  full(b3), full(w1t), full(b1), full(w2t), full(b2),
                  full(w3t), full(b3b)],
        out_specs=pl.BlockSpec((NUM_GRAPHS, OUT), lambda i: (0, 0)),
        out_shape=jax.ShapeDtypeStruct((NUM_GRAPHS, OUT), jnp.float32),
        scratch_shapes=[
            pltpu.VMEM((NUM_GRAPHS, NUM_GRAPHS), jnp.float32),
            pltpu.VMEM((NUM_GRAPHS, 1), jnp.float32),
        ],
    )(p0, p1, c0, c1, r, bcol, b3, w1t, b1, w2t, b2, w3t, b3b)


def kernel(x, edge_index, batch, W1l, b1, W1r, W2l, b2, W2r, W3l, b3, W3r,
           Wlin1, blin1, Wlin2, blin2, Wlin3, blin3):
    e = edge_index.shape[1]
    nchunk = -(-e // (NW * CH))
    nchunk = -(-nchunk // NBUF) * NBUF
    ept = nchunk * CH
    epad = NW * ept

    src = jnp.concatenate(
        [edge_index[0], jnp.zeros((epad - e,), jnp.int32)]).reshape(NW, ept)
    dst = jnp.concatenate(
        [edge_index[1],
         jnp.full((epad - e,), NPAD - 1, jnp.int32)]).reshape(NW, nchunk, CH)

    x_pad = jnp.pad(x, ((0, NPAD - N), (0, 0)))
    bcol = jnp.pad(batch, (0, NPAD - N),
                   constant_values=NUM_GRAPHS).reshape(NPAD, 1)
    zeros64 = jnp.zeros((CH, 64), jnp.float32)
    ones16 = jnp.ones((CH, CNTW), jnp.float32)
    zeros16 = jnp.zeros((CH, CNTW), jnp.float32)

    agg32 = _make_edge_agg(32, nchunk, True)
    agg48 = _make_edge_agg(48, nchunk, False)
    agg64 = _make_edge_agg(64, nchunk, False)

    t1, r1 = _stage_in(x_pad, W1l.T, W1r.T)
    p1, cnt = agg32(t1, src, dst, zeros64[:, :32], ones16, zeros16)
    c0 = cnt[0, :, 0].reshape(NPAD, 1)
    c1 = cnt[1, :, 0].reshape(NPAD, 1)
    t2, r2 = _stage_mid(p1[0], p1[1], c0, c1, r1, b1.reshape(1, 32),
                        W2l.T, W2r.T)
    (p2,) = agg48(t2, src, dst, zeros64[:, :48])
    t3, r3 = _stage_mid(p2[0], p2[1], c0, c1, r2, b2.reshape(1, 48),
                        W3l.T, W3r.T)
    (p3,) = agg64(t3, src, dst, zeros64)
    out = _stage_out(p3[0], p3[1], c0, c1, r3, bcol.astype(jnp.int32),
                     b3.reshape(1, 64), Wlin1.T, blin1.reshape(1, 32),
                     Wlin2.T, blin2.reshape(1, 32), Wlin3.T,
                     blin3.reshape(1, OUT))
    return out


# speedup vs baseline: 8.0182x; 8.0182x over previous
import jax
import jax.numpy as jnp
from jax import lax
from jax.experimental import pallas as pl
from jax.experimental.pallas import tpu as pltpu
from jax.experimental.pallas import tpu_sc as plsc

N = 10000
NUM_GRAPHS = 64
OUT = 10
NC = 2
NS = 16
NW = NC * NS
CH = 128
NBUF = 4
NPAD = 10240
RPT = NPAD // NS
CNTW = 16


def _make_edge_agg(d, nchunk, with_cnt):
    ept = nchunk * CH
    mesh = plsc.VectorSubcoreMesh(core_axis_name="c", subcore_axis_name="s")

    out_type = [jax.ShapeDtypeStruct((NC, NPAD, d), jnp.float32)]
    scratch = [
        pltpu.VMEM((ept,), jnp.int32),
        pltpu.VMEM((nchunk, CH), jnp.int32),
        pltpu.VMEM((NBUF, CH, d), jnp.float32),
        pltpu.VMEM((CH, d), jnp.float32),
        pltpu.VMEM_SHARED((NPAD, d), jnp.float32),
    ] + [pltpu.SemaphoreType.DMA] * NBUF
    if with_cnt:
        out_type.append(jax.ShapeDtypeStruct((NC, NPAD, CNTW), jnp.float32))
        scratch += [
            pltpu.VMEM((CH, CNTW), jnp.float32),
            pltpu.VMEM((CH, CNTW), jnp.float32),
            pltpu.VMEM_SHARED((NPAD, CNTW), jnp.float32),
        ]

    def body(table, srcs, dsts, zeros_hbm, *rest):
        if with_cnt:
            ones_hbm, zeros16_hbm, out_p, out_c = rest[:4]
            rest = rest[4:]
        else:
            out_p = rest[0]
            rest = rest[1:]
        src_v, dst_v, rows_v, zero_v, agg_sh = rest[:5]
        sems = rest[5:5 + NBUF]
        if with_cnt:
            ones_v, zcnt_v, cnt_sh = rest[5 + NBUF:]

        cid = lax.axis_index("c")
        sid = lax.axis_index("s")
        wid = cid * NS + sid
        row0 = sid * RPT

        pltpu.sync_copy(srcs.at[wid], src_v)
        pltpu.sync_copy(dsts.at[wid], dst_v)
        pltpu.sync_copy(zeros_hbm, zero_v)
        for j in range(RPT // CH):
            pltpu.sync_copy(zero_v, agg_sh.at[pl.ds(row0 + j * CH, CH)])
        if with_cnt:
            pltpu.sync_copy(ones_hbm, ones_v)
            pltpu.sync_copy(zeros16_hbm, zcnt_v)
            for j in range(RPT // CH):
                pltpu.sync_copy(zcnt_v, cnt_sh.at[pl.ds(row0 + j * CH, CH)])
        plsc.subcore_barrier()

        def start_gather(chunk, b):
            pltpu.async_copy(
                table.at[src_v.at[pl.ds(chunk * CH, CH)]], rows_v.at[b],
                sems[b])

        def wait_gather(b):
            pltpu.make_async_copy(
                table.at[src_v.at[pl.ds(0, CH)]], rows_v.at[b],
                sems[b]).wait()

        def scatter(chunk, b):
            pltpu.sync_copy(rows_v.at[b], agg_sh.at[dst_v.at[chunk]],
                            add=True)
            if with_cnt:
                pltpu.sync_copy(ones_v, cnt_sh.at[dst_v.at[chunk]], add=True)

        for b in range(NBUF):
            start_gather(b, b)

        @pl.loop(0, nchunk - NBUF, step=NBUF)
        def _(j0):
            for b in range(NBUF):
                wait_gather(b)
                scatter(j0 + b, b)
                start_gather(j0 + NBUF + b, b)

        for b in range(NBUF):
            wait_gather(b)
            scatter(nchunk - NBUF + b, b)

        plsc.subcore_barrier()
        pltpu.sync_copy(agg_sh.at[pl.ds(row0, RPT)],
                        out_p.at[cid, pl.ds(row0, RPT)])
        if with_cnt:
            pltpu.sync_copy(cnt_sh.at[pl.ds(row0, RPT)],
                            out_c.at[cid, pl.ds(row0, RPT)])

    return pl.kernel(body, out_type=tuple(out_type), mesh=mesh,
                     scratch_types=scratch,
                     compiler_params=pltpu.CompilerParams(
                         use_tc_tiling_on_sc=False))


BN = 2048


def _stage_in_body(x_ref, wl_ref, wr_ref, t_ref, r_ref):
    xb = x_ref[...]
    t_ref[...] = jnp.dot(xb, wl_ref[...], preferred_element_type=jnp.float32)
    r_ref[...] = jnp.dot(xb, wr_ref[...], preferred_element_type=jnp.float32)


def _stage_in(x_pad, wlt, wrt):
    din, dout = wlt.shape
    return pl.pallas_call(
        _stage_in_body,
        grid=(NPAD // BN,),
        in_specs=[
            pl.BlockSpec((BN, din), lambda i: (i, 0)),
            pl.BlockSpec((din, dout), lambda i: (0, 0)),
            pl.BlockSpec((din, dout), lambda i: (0, 0)),
        ],
        out_specs=[
            pl.BlockSpec((BN, dout), lambda i: (i, 0)),
            pl.BlockSpec((BN, dout), lambda i: (i, 0)),
        ],
        out_shape=[
            jax.ShapeDtypeStruct((NPAD, dout), jnp.float32),
            jax.ShapeDtypeStruct((NPAD, dout), jnp.float32),
        ],
    )(x_pad, wlt, wrt)


def _stage_mid_body(p0, p1, c0, c1, r, b, wl, wr, t_ref, r_ref):
    cnt = jnp.maximum(c0[...] + c1[...], 1.0)
    h = (p0[...] + p1[...]) / cnt + b[...] + r[...]
    h = jnp.maximum(h, 0.0)
    t_ref[...] = jnp.dot(h, wl[...], preferred_element_type=jnp.float32)
    r_ref[...] = jnp.dot(h, wr[...], preferred_element_type=jnp.float32)


def _stage_mid(p0, p1, c0, c1, r, bias, wlt, wrt):
    din, dout = wlt.shape
    col = pl.BlockSpec((BN, 1), lambda i: (i, 0))
    blk = pl.BlockSpec((BN, din), lambda i: (i, 0))
    return pl.pallas_call(
        _stage_mid_body,
        grid=(NPAD // BN,),
        in_specs=[blk, blk, col, col, blk,
                  pl.BlockSpec((1, din), lambda i: (0, 0)),
                  pl.BlockSpec((din, dout), lambda i: (0, 0)),
                  pl.BlockSpec((din, dout), lambda i: (0, 0))],
        out_specs=[
            pl.BlockSpec((BN, dout), lambda i: (i, 0)),
            pl.BlockSpec((BN, dout), lambda i: (i, 0)),
        ],
        out_shape=[
            jax.ShapeDtypeStruct((NPAD, dout), jnp.float32),
            jax.ShapeDtypeStruct((NPAD, dout), jnp.float32),
        ],
    )(p0, p1, c0, c1, r, bias, wlt, wrt)


def _stage_out_body(p0, p1, c0, c1, r, bcol, b3, w1, b1, w2, b2, w3, b3b,
                    out_ref, gsum, gcnt):
    i = pl.program_id(0)

    @pl.when(i == 0)
    def _():
        gsum[...] = jnp.zeros_like(gsum)
        gcnt[...] = jnp.zeros_like(gcnt)

    cnt = jnp.maximum(c0[...] + c1[...], 1.0)
    h = (p0[...] + p1[...]) / cnt + b3[...] + r[...]
    gids = lax.broadcasted_iota(jnp.int32, (1, NUM_GRAPHS), 1)
    maskt = (bcol[...] == gids).astype(jnp.float32)
    dn = (((0,), (0,)), ((), ()))
    gsum[...] += lax.dot_general(maskt, h, dn,
                                 preferred_element_type=jnp.float32)
    ones = jnp.ones((BN, 1), jnp.float32)
    gcnt[...] += lax.dot_general(maskt, ones, dn,
                                 preferred_element_type=jnp.float32)

    @pl.when(i == pl.num_programs(0) - 1)
    def _():
        g = gsum[...] / jnp.maximum(gcnt[...], 1.0)
        z = jnp.dot(g, w1[...], preferred_element_type=jnp.float32) + b1[...]
        z = jnp.maximum(z, 0.0)
        z = jnp.dot(z, w2[...], preferred_element_type=jnp.float32) + b2[...]
        z = jnp.maximum(z, 0.0)
        out_ref[...] = (jnp.dot(z, w3[...],
                                preferred_element_type=jnp.float32) + b3b[...])


def _stage_out(p0, p1, c0, c1, r, bcol, b3, w1t, b1, w2t, b2, w3t, b3b):
    din = p0.shape[1]
    col = pl.BlockSpec((BN, 1), lambda i: (i, 0))
    blk = pl.BlockSpec((BN, din), lambda i: (i, 0))

    def full(a):
        return pl.BlockSpec(a.shape, lambda i: tuple(0 for _ in a.shape))

    return pl.pallas_call(
        _stage_out_body,
        grid=(NPAD // BN,),
        in_specs=[blk, blk, col, col, blk, col,
                  full(b3), full(w1t), full(b1), full(w2t), full(b2),
                  full(w3t), full(b3b)],
        out_specs=pl.BlockSpec((NUM_GRAPHS, OUT), lambda i: (0, 0)),
        out_shape=jax.ShapeDtypeStruct((NUM_GRAPHS, OUT), jnp.float32),
        scratch_shapes=[
            pltpu.VMEM((NUM_GRAPHS, NUM_GRAPHS), jnp.float32),
            pltpu.VMEM((NUM_GRAPHS, 1), jnp.float32),
        ],
    )(p0, p1, c0, c1, r, bcol, b3, w1t, b1, w2t, b2, w3t, b3b)


def kernel(x, edge_index, batch, W1l, b1, W1r, W2l, b2, W2r, W3l, b3, W3r,
           Wlin1, blin1, Wlin2, blin2, Wlin3, blin3):
    e = edge_index.shape[1]
    nchunk = -(-e // (NW * CH))
    nchunk = -(-nchunk // NBUF) * NBUF
    ept = nchunk * CH
    epad = NW * ept

    src = jnp.concatenate(
        [edge_index[0], jnp.zeros((epad - e,), jnp.int32)]).reshape(NW, ept)
    dst = jnp.concatenate(
        [edge_index[1],
         jnp.full((epad - e,), NPAD - 1, jnp.int32)]).reshape(NW, nchunk, CH)

    x_pad = jnp.pad(x, ((0, NPAD - N), (0, 0)))
    bcol = jnp.pad(batch, (0, NPAD - N),
                   constant_values=NUM_GRAPHS).reshape(NPAD, 1)
    zeros64 = jnp.zeros((CH, 64), jnp.float32)
    ones16 = jnp.ones((CH, CNTW), jnp.float32)
    zeros16 = jnp.zeros((CH, CNTW), jnp.float32)

    agg32 = _make_edge_agg(32, nchunk, True)
    agg48 = _make_edge_agg(48, nchunk, False)
    agg64 = _make_edge_agg(64, nchunk, False)

    t1, r1 = _stage_in(x_pad, W1l.T, W1r.T)
    p1, cnt = agg32(t1, src, dst, zeros64[:, :32], ones16, zeros16)
    c0 = cnt[0, :, 0].reshape(NPAD, 1)
    c1 = cnt[1, :, 0].reshape(NPAD, 1)
    t2, r2 = _stage_mid(p1[0], p1[1], c0, c1, r1, b1.reshape(1, 32),
                        W2l.T, W2r.T)
    (p2,) = agg48(t2, src, dst, zeros64[:, :48])
    t3, r3 = _stage_mid(p2[0], p2[1], c0, c1, r2, b2.reshape(1, 48),
                        W3l.T, W3r.T)
    (p3,) = agg64(t3, src, dst, zeros64)
    out = _stage_out(p3[0], p3[1], c0, c1, r3, bcol.astype(jnp.int32),
                     b3.reshape(1, 64), Wlin1.T, blin1.reshape(1, 32),
                     Wlin2.T, blin2.reshape(1, 32), Wlin3.T,
                     blin3.reshape(1, OUT))
    return out
